# baseline (speedup 1.0000x reference)
import jax
import jax.numpy as jnp
from jax.experimental import pallas as pl
from jax.experimental.pallas import tpu as pltpu

_N = 10000
_IN = 256
_OUT = 128
_H1 = 2 * _OUT
_TILE = 2000


def _fused_encoder(x_ref, w1z_ref, b1z_ref, w1h_ref, b1h_ref,
                   w2z_ref, b2z_ref, w2h_ref, b2h_ref, o_ref,
                   w1z_s, w1h_s, w2z_s, w2h_s):
    bf16 = jnp.bfloat16
    x = x_ref[...].astype(bf16)

    @pl.when(pl.program_id(0) == 0)
    def _fold():
        w1z_s[...] = (0.5 * (w1z_ref[0, 0] + w1z_ref[1, 0])).astype(bf16)
        w1h_s[...] = (w1h_ref[0, 0] + w1h_ref[1, 0]).astype(bf16)
        w2z_s[...] = (0.25 * (w2z_ref[0, 0] + w2z_ref[1, 0])).astype(bf16)
        w2h_s[...] = (0.5 * (w2h_ref[0, 0] + w2h_ref[1, 0])).astype(bf16)

    w1z = w1z_s[...]
    w1h = w1h_s[...]
    w2z = w2z_s[...]
    w2h = w2h_s[...]
    tz1 = jnp.tanh(
        jnp.dot(x, w1z, preferred_element_type=jnp.float32)
        + 0.5 * b1z_ref[...])
    h1 = jnp.tanh(
        jnp.dot(x, w1h, preferred_element_type=jnp.float32) + b1h_ref[...])
    g = jnp.maximum((1.0 - tz1) * h1, 0.0).astype(bf16)
    tz2 = jnp.tanh(
        jnp.dot(g, w2z, preferred_element_type=jnp.float32)
        + 0.5 * b2z_ref[...])
    h2 = jnp.tanh(
        jnp.dot(g, w2h, preferred_element_type=jnp.float32) + b2h_ref[...])
    o_ref[...] = (0.5 - 0.5 * tz2) * h2


def kernel(x, edge_index, W1z, b1z, W1r, b1r, W1h, b1h,
           W2z, b2z, W2r, b2r, W2h, b2h):
    del edge_index, W1r, b1r, W2r, b2r
    grid = _N // _TILE
    row_spec = pl.BlockSpec((_TILE, _IN), lambda i: (i, 0))
    full = lambda shape: pl.BlockSpec(shape, lambda i: (0,) * len(shape))

    return pl.pallas_call(
        _fused_encoder,
        grid=(grid,),
        in_specs=[
            row_spec,
            full((2, 1, _IN, _H1)), full((_H1,)),
            full((2, 1, _IN, _H1)), full((_H1,)),
            full((2, 1, _H1, _OUT)), full((_OUT,)),
            full((2, 1, _H1, _OUT)), full((_OUT,)),
        ],
        out_specs=pl.BlockSpec((_TILE, _OUT), lambda i: (i, 0)),
        out_shape=jax.ShapeDtypeStruct((_N, _OUT), jnp.float32),
        scratch_shapes=[
            pltpu.VMEM((_IN, _H1), jnp.bfloat16),
            pltpu.VMEM((_IN, _H1), jnp.bfloat16),
            pltpu.VMEM((_H1, _OUT), jnp.bfloat16),
            pltpu.VMEM((_H1, _OUT), jnp.bfloat16),
        ],
        compiler_params=pltpu.CompilerParams(
            dimension_semantics=("arbitrary",),
            vmem_limit_bytes=100 * 1024 * 1024),
    )(x, W1z, b1z, W1h, b1h, W2z, b2z, W2h, b2h)


# speedup vs baseline: 1.0457x; 1.0457x over previous
import jax
import jax.numpy as jnp
from jax.experimental import pallas as pl
from jax.experimental.pallas import tpu as pltpu

_N = 10000
_IN = 256
_OUT = 128
_H1 = 2 * _OUT
_TILE = 2000


def _fused_encoder(x_ref, w1z_ref, b1z_ref, w1h_ref, b1h_ref,
                   w2z_ref, b2z_ref, w2h_ref, b2h_ref, o_ref):
    bf16 = jnp.bfloat16
    x = x_ref[...].astype(bf16)
    w1 = jnp.concatenate(
        [(0.5 * (w1z_ref[0, 0] + w1z_ref[1, 0])).astype(bf16),
         (w1h_ref[0, 0] + w1h_ref[1, 0]).astype(bf16)], axis=1)
    w2 = jnp.concatenate(
        [(0.25 * (w2z_ref[0, 0] + w2z_ref[1, 0])).astype(bf16),
         (0.5 * (w2h_ref[0, 0] + w2h_ref[1, 0])).astype(bf16)], axis=1)
    b1 = jnp.concatenate([0.5 * b1z_ref[...], b1h_ref[...]])
    b2 = jnp.concatenate([0.5 * b2z_ref[...], b2h_ref[...]])
    t1 = jnp.tanh(
        jnp.dot(x, w1, preferred_element_type=jnp.float32) + b1)
    g = jnp.maximum((1.0 - t1[:, :_H1]) * t1[:, _H1:], 0.0).astype(bf16)
    t2 = jnp.tanh(
        jnp.dot(g, w2, preferred_element_type=jnp.float32) + b2)
    o_ref[...] = (0.5 - 0.5 * t2[:, :_OUT]) * t2[:, _OUT:]


def kernel(x, edge_index, W1z, b1z, W1r, b1r, W1h, b1h,
           W2z, b2z, W2r, b2r, W2h, b2h):
    del edge_index, W1r, b1r, W2r, b2r
    grid = _N // _TILE
    row_spec = pl.BlockSpec((_TILE, _IN), lambda i: (i, 0))
    full = lambda shape: pl.BlockSpec(shape, lambda i: (0,) * len(shape))

    return pl.pallas_call(
        _fused_encoder,
        grid=(grid,),
        in_specs=[
            row_spec,
            full((2, 1, _IN, _H1)), full((_H1,)),
            full((2, 1, _IN, _H1)), full((_H1,)),
            full((2, 1, _H1, _OUT)), full((_OUT,)),
            full((2, 1, _H1, _OUT)), full((_OUT,)),
        ],
        out_specs=pl.BlockSpec((_TILE, _OUT), lambda i: (i, 0)),
        out_shape=jax.ShapeDtypeStruct((_N, _OUT), jnp.float32),
        compiler_params=pltpu.CompilerParams(
            dimension_semantics=("arbitrary",),
            vmem_limit_bytes=100 * 1024 * 1024),
    )(x, W1z, b1z, W1h, b1h, W2z, b2z, W2h, b2h)


# speedup vs baseline: 1.0641x; 1.0176x over previous
import jax
import jax.numpy as jnp
from jax.experimental import pallas as pl
from jax.experimental.pallas import tpu as pltpu

_N = 10000
_IN = 256
_OUT = 128
_H1 = 2 * _OUT
_TILE = 2000


def _fused_encoder(x_ref, w1z_ref, b1z_ref, w1h_ref, b1h_ref,
                   w2z_ref, b2z_ref, w2h_ref, b2h_ref, o_ref):
    x = x_ref[...]
    w1 = jnp.concatenate(
        [(0.5 * (w1z_ref[0, 0] + w1z_ref[1, 0])),
         (w1h_ref[0, 0] + w1h_ref[1, 0])], axis=1)
    w2 = jnp.concatenate(
        [(0.25 * (w2z_ref[0, 0] + w2z_ref[1, 0])),
         (0.5 * (w2h_ref[0, 0] + w2h_ref[1, 0]))], axis=1)
    b1 = jnp.concatenate([0.5 * b1z_ref[...], b1h_ref[...]])
    b2 = jnp.concatenate([0.5 * b2z_ref[...], b2h_ref[...]])
    t1 = jnp.tanh(
        jnp.dot(x, w1, preferred_element_type=jnp.float32) + b1)
    g = jnp.maximum((1.0 - t1[:, :_H1]) * t1[:, _H1:], 0.0)
    t2 = jnp.tanh(
        jnp.dot(g, w2, preferred_element_type=jnp.float32) + b2)
    o_ref[...] = (0.5 - 0.5 * t2[:, :_OUT]) * t2[:, _OUT:]


def kernel(x, edge_index, W1z, b1z, W1r, b1r, W1h, b1h,
           W2z, b2z, W2r, b2r, W2h, b2h):
    del edge_index, W1r, b1r, W2r, b2r
    grid = _N // _TILE
    row_spec = pl.BlockSpec((_TILE, _IN), lambda i: (i, 0))
    full = lambda shape: pl.BlockSpec(shape, lambda i: (0,) * len(shape))

    return pl.pallas_call(
        _fused_encoder,
        grid=(grid,),
        in_specs=[
            row_spec,
            full((2, 1, _IN, _H1)), full((_H1,)),
            full((2, 1, _IN, _H1)), full((_H1,)),
            full((2, 1, _H1, _OUT)), full((_OUT,)),
            full((2, 1, _H1, _OUT)), full((_OUT,)),
        ],
        out_specs=pl.BlockSpec((_TILE, _OUT), lambda i: (i, 0)),
        out_shape=jax.ShapeDtypeStruct((_N, _OUT), jnp.float32),
        compiler_params=pltpu.CompilerParams(
            dimension_semantics=("arbitrary",),
            vmem_limit_bytes=100 * 1024 * 1024),
    )(x, W1z, b1z, W1h, b1h, W2z, b2z, W2h, b2h)


# speedup vs baseline: 1.1845x; 1.1131x over previous
import jax
import jax.numpy as jnp
from jax.experimental import pallas as pl
from jax.experimental.pallas import tpu as pltpu

_N = 10000
_IN = 256
_OUT = 128
_H1 = 2 * _OUT
_TILE = 5000


def _fused_encoder(x_ref, w1z_ref, b1z_ref, w1h_ref, b1h_ref,
                   w2z_ref, b2z_ref, w2h_ref, b2h_ref, o_ref):
    x = x_ref[...]
    w1 = jnp.concatenate(
        [(0.5 * (w1z_ref[0, 0] + w1z_ref[1, 0])),
         (w1h_ref[0, 0] + w1h_ref[1, 0])], axis=1)
    w2 = jnp.concatenate(
        [(0.25 * (w2z_ref[0, 0] + w2z_ref[1, 0])),
         (0.5 * (w2h_ref[0, 0] + w2h_ref[1, 0]))], axis=1)
    b1 = jnp.concatenate([0.5 * b1z_ref[...], b1h_ref[...]])
    b2 = jnp.concatenate([0.5 * b2z_ref[...], b2h_ref[...]])
    t1 = jnp.tanh(
        jnp.dot(x, w1, preferred_element_type=jnp.float32) + b1)
    g = jnp.maximum((1.0 - t1[:, :_H1]) * t1[:, _H1:], 0.0)
    t2 = jnp.tanh(
        jnp.dot(g, w2, preferred_element_type=jnp.float32) + b2)
    o_ref[...] = (0.5 - 0.5 * t2[:, :_OUT]) * t2[:, _OUT:]


def kernel(x, edge_index, W1z, b1z, W1r, b1r, W1h, b1h,
           W2z, b2z, W2r, b2r, W2h, b2h):
    del edge_index, W1r, b1r, W2r, b2r
    grid = _N // _TILE
    row_spec = pl.BlockSpec((_TILE, _IN), lambda i: (i, 0))
    full = lambda shape: pl.BlockSpec(shape, lambda i: (0,) * len(shape))

    return pl.pallas_call(
        _fused_encoder,
        grid=(grid,),
        in_specs=[
            row_spec,
            full((2, 1, _IN, _H1)), full((_H1,)),
            full((2, 1, _IN, _H1)), full((_H1,)),
            full((2, 1, _H1, _OUT)), full((_OUT,)),
            full((2, 1, _H1, _OUT)), full((_OUT,)),
        ],
        out_specs=pl.BlockSpec((_TILE, _OUT), lambda i: (i, 0)),
        out_shape=jax.ShapeDtypeStruct((_N, _OUT), jnp.float32),
        compiler_params=pltpu.CompilerParams(
            dimension_semantics=("arbitrary",),
            vmem_limit_bytes=100 * 1024 * 1024),
    )(x, W1z, b1z, W1h, b1h, W2z, b2z, W2h, b2h)
